# baseline (speedup 1.0000x reference)
import numpy as np

import jax
import jax.numpy as jnp
from jax.experimental import pallas as pl
from jax.experimental.pallas import tpu as pltpu
from jax.sharding import Mesh, PartitionSpec as P

_TS = 1024
_TO = 512


def _mlora_kernel(ids_ref, x_ref, w_ref, bias_ref, a_ref, b_ref, out_ref, h_ref):
    del ids_ref
    o = pl.program_id(1)

    @pl.when(o == 0)
    def _():
        h_ref[...] = jax.lax.dot_general(
            x_ref[...], a_ref[0],
            (((1,), (0,)), ((), ())),
            preferred_element_type=jnp.float32,
        ).astype(jnp.bfloat16)

    base = jax.lax.dot_general(
        x_ref[...], w_ref[...],
        (((1,), (1,)), ((), ())),
        preferred_element_type=jnp.float32,
    )
    lora = jax.lax.dot_general(
        h_ref[...], b_ref[0],
        (((1,), (0,)), ((), ())),
        preferred_element_type=jnp.float32,
    )
    out_ref[...] = base + lora + bias_ref[...]


def _mlora_shard(tile_ids, xb, Wb, bias2, Ab, Bb):
    BS, D = xb.shape
    O = Wb.shape[0]
    L, _, R = Ab.shape
    ts = min(_TS, BS)
    to = min(_TO, O)
    n_t, n_o = BS // ts, O // to

    grid_spec = pltpu.PrefetchScalarGridSpec(
        num_scalar_prefetch=1,
        grid=(n_t, n_o),
        in_specs=[
            pl.BlockSpec((ts, D), lambda t, o, ids: (t, 0)),
            pl.BlockSpec((to, D), lambda t, o, ids: (o, 0)),
            pl.BlockSpec((1, to), lambda t, o, ids: (0, o)),
            pl.BlockSpec((1, D, R), lambda t, o, ids: (ids[t], 0, 0)),
            pl.BlockSpec((1, R, to), lambda t, o, ids: (ids[t], 0, o)),
        ],
        out_specs=pl.BlockSpec((ts, to), lambda t, o, ids: (t, o)),
        scratch_shapes=[pltpu.VMEM((ts, R), jnp.bfloat16)],
    )

    return pl.pallas_call(
        _mlora_kernel,
        grid_spec=grid_spec,
        out_shape=jax.ShapeDtypeStruct((BS, O), jnp.float32),
        compiler_params=pltpu.CompilerParams(
            dimension_semantics=("parallel", "arbitrary"),
        ),
    )(tile_ids, xb, Wb, bias2, Ab, Bb)


def kernel(x, adapter_ids, W, bias, lora_A, lora_B):
    Bn, S, D = x.shape
    O = W.shape[0]
    BS = Bn * S

    xb = x.reshape(BS, D).astype(jnp.bfloat16)
    Wb = W.astype(jnp.bfloat16)
    Ab = lora_A.astype(jnp.bfloat16)
    Bb = lora_B.astype(jnp.bfloat16)
    bias2 = bias.reshape(1, O)
    ts = min(_TS, S)
    tile_ids = jnp.repeat(adapter_ids.astype(jnp.int32), S // ts)

    devs = jax.devices()
    n_tp = 2 if (len(devs) >= 2 and O % (2 * _TO) == 0) else 1
    mesh = Mesh(np.array(devs[:n_tp]), ("tp",))
    shard_fn = jax.shard_map(
        _mlora_shard,
        mesh=mesh,
        in_specs=(P(), P(), P("tp", None), P(None, "tp"), P(), P(None, None, "tp")),
        out_specs=P(None, "tp"),
        check_vma=False,
    )
    out = shard_fn(tile_ids, xb, Wb, bias2, Ab, Bb)
    return out.reshape(Bn, S, O)


# speedup vs baseline: 1.9612x; 1.9612x over previous
import jax
import jax.numpy as jnp
from jax.experimental import pallas as pl
from jax.experimental.pallas import tpu as pltpu

_TS = 1024
_TO = 256


def _mlora_kernel(ids_ref, x_ref, w_ref, bias_ref, a_ref, b_ref, out_ref,
                  h_ref, xb_ref):
    del ids_ref
    o = pl.program_id(1)

    @pl.when(o == 0)
    def _():
        xb_ref[...] = x_ref[...].astype(jnp.bfloat16)
        h_ref[...] = jax.lax.dot_general(
            xb_ref[...], a_ref[0].astype(jnp.bfloat16),
            (((1,), (0,)), ((), ())),
            preferred_element_type=jnp.float32,
        ).astype(jnp.bfloat16)

    base = jax.lax.dot_general(
        xb_ref[...], w_ref[...].astype(jnp.bfloat16),
        (((1,), (1,)), ((), ())),
        preferred_element_type=jnp.float32,
    )
    lora = jax.lax.dot_general(
        h_ref[...], b_ref[0].astype(jnp.bfloat16),
        (((1,), (0,)), ((), ())),
        preferred_element_type=jnp.float32,
    )
    out_ref[...] = base + lora + bias_ref[...]


def kernel(x, adapter_ids, W, bias, lora_A, lora_B):
    Bn, S, D = x.shape
    O = W.shape[0]
    L, _, R = lora_A.shape
    BS = Bn * S
    ts = min(_TS, S)
    to = min(_TO, O)
    n_t, n_o = BS // ts, O // to

    x2 = x.reshape(BS, D)
    bias2 = bias.reshape(1, O)
    tile_ids = jnp.repeat(adapter_ids.astype(jnp.int32), S // ts)

    grid_spec = pltpu.PrefetchScalarGridSpec(
        num_scalar_prefetch=1,
        grid=(n_t, n_o),
        in_specs=[
            pl.BlockSpec((ts, D), lambda t, o, ids: (t, 0)),
            pl.BlockSpec((to, D), lambda t, o, ids: (o, 0)),
            pl.BlockSpec((1, to), lambda t, o, ids: (0, o)),
            pl.BlockSpec((1, D, R), lambda t, o, ids: (ids[t], 0, 0)),
            pl.BlockSpec((1, R, to), lambda t, o, ids: (ids[t], 0, o)),
        ],
        out_specs=pl.BlockSpec((ts, to), lambda t, o, ids: (t, o)),
        scratch_shapes=[
            pltpu.VMEM((ts, R), jnp.bfloat16),
            pltpu.VMEM((ts, D), jnp.bfloat16),
        ],
    )

    out = pl.pallas_call(
        _mlora_kernel,
        grid_spec=grid_spec,
        out_shape=jax.ShapeDtypeStruct((BS, O), jnp.float32),
        compiler_params=pltpu.CompilerParams(
            dimension_semantics=("parallel", "arbitrary"),
        ),
    )(tile_ids, x2, W, bias2, lora_A, lora_B)
    return out.reshape(Bn, S, O)


# speedup vs baseline: 2.0939x; 1.0677x over previous
import jax
import jax.numpy as jnp
from jax.experimental import pallas as pl
from jax.experimental.pallas import tpu as pltpu

_TS = 1024
_TO = 512


def _mlora_kernel(ids_ref, x_ref, w_ref, bias_ref, a_ref, b_ref, out_ref,
                  h_ref):
    del ids_ref
    o = pl.program_id(1)

    @pl.when(o == 0)
    def _():
        h_ref[...] = jax.lax.dot_general(
            x_ref[...].astype(jnp.bfloat16), a_ref[0].astype(jnp.bfloat16),
            (((1,), (0,)), ((), ())),
            preferred_element_type=jnp.float32,
        ).astype(jnp.bfloat16)

    base = jax.lax.dot_general(
        x_ref[...].astype(jnp.bfloat16), w_ref[...].astype(jnp.bfloat16),
        (((1,), (1,)), ((), ())),
        preferred_element_type=jnp.float32,
    )
    lora = jax.lax.dot_general(
        h_ref[...], b_ref[0].astype(jnp.bfloat16),
        (((1,), (0,)), ((), ())),
        preferred_element_type=jnp.float32,
    )
    out_ref[...] = base + lora + bias_ref[...]


def kernel(x, adapter_ids, W, bias, lora_A, lora_B):
    Bn, S, D = x.shape
    O = W.shape[0]
    L, _, R = lora_A.shape
    BS = Bn * S
    ts = min(_TS, S)
    to = min(_TO, O)
    n_t, n_o = BS // ts, O // to

    x2 = x.reshape(BS, D)
    bias2 = bias.reshape(1, O)
    tile_ids = jnp.repeat(adapter_ids.astype(jnp.int32), S // ts)

    grid_spec = pltpu.PrefetchScalarGridSpec(
        num_scalar_prefetch=1,
        grid=(n_t, n_o),
        in_specs=[
            pl.BlockSpec((ts, D), lambda t, o, ids: (t, 0)),
            pl.BlockSpec((to, D), lambda t, o, ids: (o, 0)),
            pl.BlockSpec((1, to), lambda t, o, ids: (0, o)),
            pl.BlockSpec((1, D, R), lambda t, o, ids: (ids[t], 0, 0)),
            pl.BlockSpec((1, R, to), lambda t, o, ids: (ids[t], 0, o)),
        ],
        out_specs=pl.BlockSpec((ts, to), lambda t, o, ids: (t, o)),
        scratch_shapes=[
            pltpu.VMEM((ts, R), jnp.bfloat16),
        ],
    )

    out = pl.pallas_call(
        _mlora_kernel,
        grid_spec=grid_spec,
        out_shape=jax.ShapeDtypeStruct((BS, O), jnp.float32),
        compiler_params=pltpu.CompilerParams(
            dimension_semantics=("parallel", "arbitrary"),
        ),
    )(tile_ids, x2, W, bias2, lora_A, lora_B)
    return out.reshape(Bn, S, O)
